# baseline (speedup 1.0000x reference)
import functools

import jax
import jax.numpy as jnp
from jax import lax
from jax.experimental import pallas as pl
from jax.experimental.pallas import tpu as pltpu
from jax.experimental.pallas import tpu_sc as plsc


def _oem_pairs(n):
    pairs = []
    p = 1
    while p < n:
        k = p
        while k >= 1:
            for j in range(k % p, n - k, 2 * k):
                for i in range(0, min(k, n - j - k)):
                    if (i + j) // (2 * p) == (i + j + k) // (2 * p):
                        pairs.append((i + j, i + j + k))
            k //= 2
        p *= 2
    return pairs


_PAIRS16 = _oem_pairs(16)


def _tree_reduce(vals, op):
    vals = list(vals)
    while len(vals) > 1:
        nxt = [op(vals[i], vals[i + 1]) for i in range(0, len(vals) - 1, 2)]
        if len(vals) % 2:
            nxt.append(vals[-1])
        vals = nxt
    return vals[0]


def _sel15_16(vals):
    a = list(vals[:16])
    b = list(vals[16:32])
    for i, j in _PAIRS16:
        lo = jnp.minimum(a[i], a[j])
        a[j] = jnp.maximum(a[i], a[j])
        a[i] = lo
        lo = jnp.minimum(b[i], b[j])
        b[j] = jnp.maximum(b[i], b[j])
        b[i] = lo
    lo = [jnp.minimum(a[i], b[15 - i]) for i in range(16)]
    hi = [jnp.maximum(a[i], b[15 - i]) for i in range(16)]
    return _tree_reduce(lo, jnp.maximum), _tree_reduce(hi, jnp.minimum)


def _median33(vals):
    v15, v16 = _sel15_16(vals[:32])
    return jnp.minimum(jnp.maximum(vals[32], v15), v16)


def _sub_iota(d):
    return jax.lax.broadcasted_iota(jnp.int32, (8, d), 0)


def _transpose8(a, d):
    sub = _sub_iota(d)
    v = list(a)
    for k in (4, 2, 1):
        mask = (sub & k) == 0
        nxt = list(v)
        for i in range(8):
            if i & k:
                continue
            j = i + k
            nxt[i] = jnp.where(mask, v[i], pltpu.roll(v[j], k, 0))
            nxt[j] = jnp.where(mask, pltpu.roll(v[i], -k % 8, 0), v[j])
        v = nxt
    return v


def _tc_body(x_ref, nb_ref, w_ref, b_ref, o_ref, *, bn, s):
    d = x_ref.shape[1]
    meds = []
    for c in range(bn // 8):
        tiles = [nb_ref[pl.ds(c * 8 * s + 8 * m, 8), :] for m in range(s)]
        vals = []
        for t in range(4):
            vals.extend(_transpose8([tiles[4 * r + t] for r in range(8)], d))
        vals.append(x_ref[pl.ds(c * 8, 8), :])
        meds.append(_median33(vals))
    med = jnp.concatenate(meds, axis=0)
    o_ref[...] = (
        jnp.dot(med, w_ref[...], preferred_element_type=jnp.float32) + b_ref[...]
    )


def _sc_body(nb_hbm, med_hbm, nbuf, obuf, *, ntc, npw, batch, s, d):
    nc = 2
    wid = lax.axis_index("s") * nc + lax.axis_index("c")
    base_node = ntc + wid * npw
    groups = d // 16
    nbatches = npw // batch

    def do_batch(b, _):
        row0 = (base_node + b * batch) * s
        pltpu.sync_copy(nb_hbm.at[pl.ds(row0, batch * s)], nbuf)

        def do_ng(i, _):
            node = i // groups
            g = i % groups
            lanes = pl.ds(g * 16, 16)
            vals = [nbuf[node * s + j, lanes] for j in range(s)]
            v15, v16 = _sel15_16(vals)
            orow = b * batch + node
            obuf[orow, pl.ds(g * 16, 16)] = v15
            obuf[orow, pl.ds(d + g * 16, 16)] = v16
            return 0

        lax.fori_loop(0, batch * groups, do_ng, 0)
        return 0

    lax.fori_loop(0, nbatches, do_batch, 0)
    pltpu.sync_copy(obuf, med_hbm.at[pl.ds(wid * npw, npw)])


def _post_body(m_ref, x_ref, w_ref, b_ref, o_ref):
    d = x_ref.shape[1]
    v15 = m_ref[:, :d]
    v16 = m_ref[:, d:]
    med = jnp.minimum(jnp.maximum(x_ref[...], v15), v16)
    o_ref[...] = (
        jnp.dot(med, w_ref[...], preferred_element_type=jnp.float32) + b_ref[...]
    )


def kernel(x, neigh_x, kernel, bias):
    n, s, d = neigh_x.shape
    units = kernel.shape[1]
    assert s == 32, "median network is specialized to 32 neighbors + self"

    nsc = 3840
    ntc = n - nsc
    bn = 560
    npw = nsc // 32
    batch = 4
    assert ntc % bn == 0 and npw % batch == 0

    bias2 = bias.reshape(1, units)
    nb_flat = neigh_x.reshape(n * s, d)

    tc_body = functools.partial(_tc_body, bn=bn, s=s)
    out_tc = pl.pallas_call(
        tc_body,
        grid=(ntc // bn,),
        in_specs=[
            pl.BlockSpec((bn, d), lambda i: (i, 0)),
            pl.BlockSpec((bn * s, d), lambda i: (i, 0)),
            pl.BlockSpec((d, units), lambda i: (0, 0)),
            pl.BlockSpec((1, units), lambda i: (0, 0)),
        ],
        out_specs=pl.BlockSpec((bn, units), lambda i: (i, 0)),
        out_shape=jax.ShapeDtypeStruct((ntc, units), jnp.float32),
    )(x, nb_flat, kernel, bias2)

    sc_body = functools.partial(_sc_body, ntc=ntc, npw=npw, batch=batch, s=s, d=d)
    med_sc = pl.kernel(
        sc_body,
        out_type=jax.ShapeDtypeStruct((nsc, 2 * d), jnp.float32),
        mesh=plsc.VectorSubcoreMesh(
            core_axis_name="c", subcore_axis_name="s", num_cores=2, num_subcores=16
        ),
        scratch_types=[
            pltpu.VMEM((batch * s, d), jnp.float32),
            pltpu.VMEM((npw, 2 * d), jnp.float32),
        ],
    )(nb_flat)

    bm = 80
    assert nsc % bm == 0 and ntc % bm == 0
    off = ntc // bm
    out_sc = pl.pallas_call(
        _post_body,
        grid=(nsc // bm,),
        in_specs=[
            pl.BlockSpec((bm, 2 * d), lambda i: (i, 0)),
            pl.BlockSpec((bm, d), lambda i: (i + off, 0)),
            pl.BlockSpec((d, units), lambda i: (0, 0)),
            pl.BlockSpec((1, units), lambda i: (0, 0)),
        ],
        out_specs=pl.BlockSpec((bm, units), lambda i: (i, 0)),
        out_shape=jax.ShapeDtypeStruct((nsc, units), jnp.float32),
    )(med_sc, x, kernel, bias2)

    return jnp.concatenate([out_tc, out_sc], axis=0)


# speedup vs baseline: 1.3516x; 1.3516x over previous
import functools

import jax
import jax.numpy as jnp
from jax import lax
from jax.experimental import pallas as pl
from jax.experimental.pallas import tpu as pltpu
from jax.experimental.pallas import tpu_sc as plsc


def _oem_pairs(n):
    pairs = []
    p = 1
    while p < n:
        k = p
        while k >= 1:
            for j in range(k % p, n - k, 2 * k):
                for i in range(0, min(k, n - j - k)):
                    if (i + j) // (2 * p) == (i + j + k) // (2 * p):
                        pairs.append((i + j, i + j + k))
            k //= 2
        p *= 2
    return pairs


_PAIRS16 = _oem_pairs(16)


def _tree_reduce(vals, op):
    vals = list(vals)
    while len(vals) > 1:
        nxt = [op(vals[i], vals[i + 1]) for i in range(0, len(vals) - 1, 2)]
        if len(vals) % 2:
            nxt.append(vals[-1])
        vals = nxt
    return vals[0]


def _sel15_16(vals):
    a = list(vals[:16])
    b = list(vals[16:32])
    for i, j in _PAIRS16:
        lo = jnp.minimum(a[i], a[j])
        a[j] = jnp.maximum(a[i], a[j])
        a[i] = lo
        lo = jnp.minimum(b[i], b[j])
        b[j] = jnp.maximum(b[i], b[j])
        b[i] = lo
    lo = [jnp.minimum(a[i], b[15 - i]) for i in range(16)]
    hi = [jnp.maximum(a[i], b[15 - i]) for i in range(16)]
    return _tree_reduce(lo, jnp.maximum), _tree_reduce(hi, jnp.minimum)


def _median33(vals):
    v15, v16 = _sel15_16(vals[:32])
    return jnp.minimum(jnp.maximum(vals[32], v15), v16)


def _sub_iota(d):
    return jax.lax.broadcasted_iota(jnp.int32, (8, d), 0)


def _transpose8(a, d):
    sub = _sub_iota(d)
    v = list(a)
    for k in (4, 2, 1):
        mask = (sub & k) == 0
        nxt = list(v)
        for i in range(8):
            if i & k:
                continue
            j = i + k
            nxt[i] = jnp.where(mask, v[i], pltpu.roll(v[j], k, 0))
            nxt[j] = jnp.where(mask, pltpu.roll(v[i], -k % 8, 0), v[j])
        v = nxt
    return v


def _tc_body(x_ref, nb_ref, w_ref, b_ref, o_ref, *, bn, s):
    d = x_ref.shape[1]
    meds = []
    for c in range(bn // 8):
        tiles = [nb_ref[pl.ds(c * 8 * s + 8 * m, 8), :] for m in range(s)]
        vals = []
        for t in range(4):
            vals.extend(_transpose8([tiles[4 * r + t] for r in range(8)], d))
        vals.append(x_ref[pl.ds(c * 8, 8), :])
        meds.append(_median33(vals))
    med = jnp.concatenate(meds, axis=0)
    o_ref[...] = (
        jnp.dot(med, w_ref[...], preferred_element_type=jnp.float32) + b_ref[...]
    )


def _sc_body(nb_hbm, med_hbm, nbuf, obuf, sem, *, ntc, npw, batch, s, d):
    nc = 2
    wid = lax.axis_index("s") * nc + lax.axis_index("c")
    base_node = ntc + wid * npw
    groups = d // 16
    nbatches = npw // batch
    rows = batch * s

    def copy_for(b):
        row0 = (base_node + b * batch) * s
        return pltpu.make_async_copy(
            nb_hbm.at[pl.ds(row0, rows)], nbuf.at[b % 2], sem
        )

    copy_for(0).start()
    copy_for(1).start()

    def do_batch(b, _):
        copy_for(b).wait()

        def do_ng(i, _):
            node = i // (groups // 2)
            g2 = i % (groups // 2)
            orow = b * batch + node
            for u in range(2):
                g = g2 * 2 + u
                vals = [nbuf[b % 2, node * s + j, pl.ds(g * 16, 16)] for j in range(s)]
                v15, v16 = _sel15_16(vals)
                obuf[orow, pl.ds(g * 16, 16)] = v15
                obuf[orow, pl.ds(d + g * 16, 16)] = v16
            return 0

        lax.fori_loop(0, batch * (groups // 2), do_ng, 0)

        @pl.when(b + 2 < nbatches)
        def _():
            copy_for(b + 2).start()

        return 0

    lax.fori_loop(0, nbatches, do_batch, 0)
    pltpu.sync_copy(obuf, med_hbm.at[pl.ds(wid * npw, npw)])


def _post_body(m_ref, x_ref, w_ref, b_ref, o_ref):
    d = x_ref.shape[1]
    v15 = m_ref[:, :d]
    v16 = m_ref[:, d:]
    med = jnp.minimum(jnp.maximum(x_ref[...], v15), v16)
    o_ref[...] = (
        jnp.dot(med, w_ref[...], preferred_element_type=jnp.float32) + b_ref[...]
    )


def kernel(x, neigh_x, kernel, bias):
    n, s, d = neigh_x.shape
    units = kernel.shape[1]
    assert s == 32, "median network is specialized to 32 neighbors + self"

    nsc = 3840
    ntc = n - nsc
    bn = 560
    npw = nsc // 32
    batch = 4
    assert ntc % bn == 0 and npw % batch == 0

    bias2 = bias.reshape(1, units)
    nb_flat = neigh_x.reshape(n * s, d)

    sc_body = functools.partial(_sc_body, ntc=ntc, npw=npw, batch=batch, s=s, d=d)
    med_sc = pl.kernel(
        sc_body,
        out_type=jax.ShapeDtypeStruct((nsc, 2 * d), jnp.float32),
        mesh=plsc.VectorSubcoreMesh(
            core_axis_name="c", subcore_axis_name="s", num_cores=2, num_subcores=16
        ),
        scratch_types=[
            pltpu.VMEM((2, batch * s, d), jnp.float32),
            pltpu.VMEM((npw, 2 * d), jnp.float32),
            pltpu.SemaphoreType.DMA,
        ],
    )(nb_flat)

    tc_body = functools.partial(_tc_body, bn=bn, s=s)
    out_tc = pl.pallas_call(
        tc_body,
        grid=(ntc // bn,),
        in_specs=[
            pl.BlockSpec((bn, d), lambda i: (i, 0)),
            pl.BlockSpec((bn * s, d), lambda i: (i, 0)),
            pl.BlockSpec((d, units), lambda i: (0, 0)),
            pl.BlockSpec((1, units), lambda i: (0, 0)),
        ],
        out_specs=pl.BlockSpec((bn, units), lambda i: (i, 0)),
        out_shape=jax.ShapeDtypeStruct((ntc, units), jnp.float32),
    )(x, nb_flat, kernel, bias2)

    bm = 80
    assert nsc % bm == 0 and ntc % bm == 0
    off = ntc // bm
    out_sc = pl.pallas_call(
        _post_body,
        grid=(nsc // bm,),
        in_specs=[
            pl.BlockSpec((bm, 2 * d), lambda i: (i, 0)),
            pl.BlockSpec((bm, d), lambda i: (i + off, 0)),
            pl.BlockSpec((d, units), lambda i: (0, 0)),
            pl.BlockSpec((1, units), lambda i: (0, 0)),
        ],
        out_specs=pl.BlockSpec((bm, units), lambda i: (i, 0)),
        out_shape=jax.ShapeDtypeStruct((nsc, units), jnp.float32),
    )(med_sc, x, kernel, bias2)

    return jnp.concatenate([out_tc, out_sc], axis=0)


# speedup vs baseline: 1.3536x; 1.0014x over previous
import functools

import jax
import jax.numpy as jnp
from jax import lax
from jax.experimental import pallas as pl
from jax.experimental.pallas import tpu as pltpu
from jax.experimental.pallas import tpu_sc as plsc


def _oem_pairs(n):
    pairs = []
    p = 1
    while p < n:
        k = p
        while k >= 1:
            for j in range(k % p, n - k, 2 * k):
                for i in range(0, min(k, n - j - k)):
                    if (i + j) // (2 * p) == (i + j + k) // (2 * p):
                        pairs.append((i + j, i + j + k))
            k //= 2
        p *= 2
    return pairs


_PAIRS16 = _oem_pairs(16)


def _tree_reduce(vals, op):
    vals = list(vals)
    while len(vals) > 1:
        nxt = [op(vals[i], vals[i + 1]) for i in range(0, len(vals) - 1, 2)]
        if len(vals) % 2:
            nxt.append(vals[-1])
        vals = nxt
    return vals[0]


def _sel15_16(vals):
    a = list(vals[:16])
    b = list(vals[16:32])
    for i, j in _PAIRS16:
        lo = jnp.minimum(a[i], a[j])
        a[j] = jnp.maximum(a[i], a[j])
        a[i] = lo
        lo = jnp.minimum(b[i], b[j])
        b[j] = jnp.maximum(b[i], b[j])
        b[i] = lo
    lo = [jnp.minimum(a[i], b[15 - i]) for i in range(16)]
    hi = [jnp.maximum(a[i], b[15 - i]) for i in range(16)]
    return _tree_reduce(lo, jnp.maximum), _tree_reduce(hi, jnp.minimum)


def _median33(vals):
    v15, v16 = _sel15_16(vals[:32])
    return jnp.minimum(jnp.maximum(vals[32], v15), v16)


def _sub_iota(d):
    return jax.lax.broadcasted_iota(jnp.int32, (8, d), 0)


def _transpose8(a, d):
    sub = _sub_iota(d)
    v = list(a)
    for k in (4, 2, 1):
        mask = (sub & k) == 0
        nxt = list(v)
        for i in range(8):
            if i & k:
                continue
            j = i + k
            nxt[i] = jnp.where(mask, v[i], pltpu.roll(v[j], k, 0))
            nxt[j] = jnp.where(mask, pltpu.roll(v[i], -k % 8, 0), v[j])
        v = nxt
    return v


def _tc_body(x_ref, nb_ref, w_ref, b_ref, o_ref, *, bn, s):
    d = x_ref.shape[1]
    meds = []
    for c in range(bn // 8):
        tiles = [nb_ref[pl.ds(c * 8 * s + 8 * m, 8), :] for m in range(s)]
        vals = []
        for t in range(4):
            vals.extend(_transpose8([tiles[4 * r + t] for r in range(8)], d))
        vals.append(x_ref[pl.ds(c * 8, 8), :])
        meds.append(_median33(vals))
    med = jnp.concatenate(meds, axis=0)
    o_ref[...] = (
        jnp.dot(med, w_ref[...], preferred_element_type=jnp.float32) + b_ref[...]
    )


def _sc_body(nb_hbm, med_hbm, nbuf, obuf, sem, *, ntc, npw, batch, s, d):
    nc = 2
    wid = lax.axis_index("s") * nc + lax.axis_index("c")
    base_node = ntc + wid * npw
    groups = d // 16
    nbatches = npw // batch
    rows = batch * s

    def copy_for(b):
        row0 = (base_node + b * batch) * s
        return pltpu.make_async_copy(
            nb_hbm.at[pl.ds(row0, rows)], nbuf.at[b % 2], sem
        )

    copy_for(0).start()
    copy_for(1).start()

    def do_batch(b, _):
        copy_for(b).wait()

        def do_ng(i, _):
            node = i // (groups // 2)
            g2 = i % (groups // 2)
            orow = b * batch + node
            for u in range(2):
                g = g2 * 2 + u
                vals = [nbuf[b % 2, node * s + j, pl.ds(g * 16, 16)] for j in range(s)]
                v15, v16 = _sel15_16(vals)
                obuf[orow, pl.ds(g * 16, 16)] = v15
                obuf[orow, pl.ds(d + g * 16, 16)] = v16
            return 0

        lax.fori_loop(0, batch * (groups // 2), do_ng, 0)

        @pl.when(b + 2 < nbatches)
        def _():
            copy_for(b + 2).start()

        return 0

    lax.fori_loop(0, nbatches, do_batch, 0)
    pltpu.sync_copy(obuf, med_hbm.at[pl.ds(wid * npw, npw)])


def _post_body(m_ref, x_ref, w_ref, b_ref, o_ref):
    d = x_ref.shape[1]
    v15 = m_ref[:, :d]
    v16 = m_ref[:, d:]
    med = jnp.minimum(jnp.maximum(x_ref[...], v15), v16)
    o_ref[...] = (
        jnp.dot(med, w_ref[...], preferred_element_type=jnp.float32) + b_ref[...]
    )


def kernel(x, neigh_x, kernel, bias):
    n, s, d = neigh_x.shape
    units = kernel.shape[1]
    assert s == 32, "median network is specialized to 32 neighbors + self"

    nsc = 3840
    ntc = n - nsc
    bn = 560
    npw = nsc // 32
    batch = 4
    assert ntc % bn == 0 and npw % batch == 0

    bias2 = bias.reshape(1, units)
    nb_flat = neigh_x.reshape(n * s, d)

    sc_body = functools.partial(_sc_body, ntc=ntc, npw=npw, batch=batch, s=s, d=d)
    med_sc = pl.kernel(
        sc_body,
        out_type=jax.ShapeDtypeStruct((nsc, 2 * d), jnp.float32),
        mesh=plsc.VectorSubcoreMesh(
            core_axis_name="c", subcore_axis_name="s", num_cores=2, num_subcores=16
        ),
        scratch_types=[
            pltpu.VMEM((2, batch * s, d), jnp.float32),
            pltpu.VMEM((npw, 2 * d), jnp.float32),
            pltpu.SemaphoreType.DMA,
        ],
        cost_estimate=pl.CostEstimate(
            flops=nsc * d * 400,
            transcendentals=0,
            bytes_accessed=nsc * s * d * 4,
        ),
    )(nb_flat)

    tc_body = functools.partial(_tc_body, bn=bn, s=s)
    out_tc = pl.pallas_call(
        tc_body,
        grid=(ntc // bn,),
        in_specs=[
            pl.BlockSpec((bn, d), lambda i: (i, 0)),
            pl.BlockSpec((bn * s, d), lambda i: (i, 0)),
            pl.BlockSpec((d, units), lambda i: (0, 0)),
            pl.BlockSpec((1, units), lambda i: (0, 0)),
        ],
        out_specs=pl.BlockSpec((bn, units), lambda i: (i, 0)),
        out_shape=jax.ShapeDtypeStruct((ntc, units), jnp.float32),
    )(x, nb_flat, kernel, bias2)

    bm = 80
    assert nsc % bm == 0 and ntc % bm == 0
    off = ntc // bm
    out_sc = pl.pallas_call(
        _post_body,
        grid=(nsc // bm,),
        in_specs=[
            pl.BlockSpec((bm, 2 * d), lambda i: (i, 0)),
            pl.BlockSpec((bm, d), lambda i: (i + off, 0)),
            pl.BlockSpec((d, units), lambda i: (0, 0)),
            pl.BlockSpec((1, units), lambda i: (0, 0)),
        ],
        out_specs=pl.BlockSpec((bm, units), lambda i: (i, 0)),
        out_shape=jax.ShapeDtypeStruct((nsc, units), jnp.float32),
    )(med_sc, x, kernel, bias2)

    return jnp.concatenate([out_tc, out_sc], axis=0)


# speedup vs baseline: 1.8940x; 1.3993x over previous
import jax
import jax.numpy as jnp
from jax.experimental import pallas as pl
from jax.experimental.pallas import tpu as pltpu


def _oem_pairs(n):
    pairs = []
    p = 1
    while p < n:
        k = p
        while k >= 1:
            for j in range(k % p, n - k, 2 * k):
                for i in range(0, min(k, n - j - k)):
                    if (i + j) // (2 * p) == (i + j + k) // (2 * p):
                        pairs.append((i + j, i + j + k))
            k //= 2
        p *= 2
    return pairs


_PAIRS16 = _oem_pairs(16)


def _tree_reduce(vals, op):
    vals = list(vals)
    while len(vals) > 1:
        nxt = [op(vals[i], vals[i + 1]) for i in range(0, len(vals) - 1, 2)]
        if len(vals) % 2:
            nxt.append(vals[-1])
        vals = nxt
    return vals[0]


def _median33(vals):
    a = list(vals[:16])
    b = list(vals[16:32])
    e = vals[32]
    for i, j in _PAIRS16:
        lo = jnp.minimum(a[i], a[j])
        a[j] = jnp.maximum(a[i], a[j])
        a[i] = lo
        lo = jnp.minimum(b[i], b[j])
        b[j] = jnp.maximum(b[i], b[j])
        b[i] = lo
    lo = [jnp.minimum(a[i], b[15 - i]) for i in range(16)]
    hi = [jnp.maximum(a[i], b[15 - i]) for i in range(16)]
    v15 = _tree_reduce(lo, jnp.maximum)
    v16 = _tree_reduce(hi, jnp.minimum)
    return jnp.minimum(jnp.maximum(e, v15), v16)


def _sub_iota(d):
    return jax.lax.broadcasted_iota(jnp.int32, (8, d), 0)


def _transpose8(a, d):
    sub = _sub_iota(d)
    v = list(a)
    for k in (4, 2, 1):
        mask = (sub & k) == 0
        nxt = list(v)
        for i in range(8):
            if i & k:
                continue
            j = i + k
            nxt[i] = jnp.where(mask, v[i], pltpu.roll(v[j], k, 0))
            nxt[j] = jnp.where(mask, pltpu.roll(v[i], -k % 8, 0), v[j])
        v = nxt
    return v


def _body(x_ref, nb_ref, w_ref, b_ref, o_ref, *, bn, s):
    d = x_ref.shape[1]
    meds = []
    for c in range(bn // 8):
        tiles = [nb_ref[pl.ds(c * 8 * s + 8 * m, 8), :] for m in range(s)]
        vals = []
        for t in range(4):
            vals.extend(_transpose8([tiles[4 * r + t] for r in range(8)], d))
        vals.append(x_ref[pl.ds(c * 8, 8), :])
        meds.append(_median33(vals))
    med = jnp.concatenate(meds, axis=0)
    o_ref[...] = (
        jnp.dot(med, w_ref[...], preferred_element_type=jnp.float32) + b_ref[...]
    )


def kernel(x, neigh_x, kernel, bias):
    n, s, d = neigh_x.shape
    units = kernel.shape[1]
    assert s == 32, "median network is specialized to 32 neighbors + self"
    bn = 400
    assert n % bn == 0
    bias2 = bias.reshape(1, units)
    nb_flat = neigh_x.reshape(n * s, d)

    import functools

    body = functools.partial(_body, bn=bn, s=s)
    return pl.pallas_call(
        body,
        grid=(n // bn,),
        in_specs=[
            pl.BlockSpec((bn, d), lambda i: (i, 0)),
            pl.BlockSpec((bn * s, d), lambda i: (i, 0)),
            pl.BlockSpec((d, units), lambda i: (0, 0)),
            pl.BlockSpec((1, units), lambda i: (0, 0)),
        ],
        out_specs=pl.BlockSpec((bn, units), lambda i: (i, 0)),
        out_shape=jax.ShapeDtypeStruct((n, units), jnp.float32),
    )(x, nb_flat, kernel, bias2)
